# Initial kernel scaffold; baseline (speedup 1.0000x reference)
import jax
import jax.numpy as jnp
from jax.experimental import pallas as pl


def kernel(embedding, indices):
    raise NotImplementedError("write your pallas kernel here")


# speedup vs baseline: 1.5454x; 1.5454x over previous
import functools

import jax
import jax.numpy as jnp
from jax import lax
from jax.experimental import pallas as pl
from jax.experimental.pallas import tpu as pltpu
from jax.experimental.pallas import tpu_sc as plsc

_VOCAB = 1000000
_EMBED_DIM = 32
_BATCH = 16384
_FIELDS = 26
_TOTAL = _BATCH * _FIELDS
_NW = 32
_PER_W = _TOTAL // _NW
_CHUNK = 1024
_NCHUNK = _PER_W // _CHUNK

_mesh = plsc.VectorSubcoreMesh(core_axis_name="c", subcore_axis_name="s")


@functools.partial(
    pl.kernel,
    out_type=jax.ShapeDtypeStruct((_TOTAL, _EMBED_DIM), jnp.float32),
    mesh=_mesh,
    scratch_types=[
        pltpu.VMEM((_CHUNK,), jnp.int32),
        pltpu.VMEM((_CHUNK, _EMBED_DIM), jnp.float32),
        pltpu.SemaphoreType.DMA,
    ],
    compiler_params=pltpu.CompilerParams(use_tc_tiling_on_sc=False),
)
def _gather(table_hbm, idx_hbm, out_hbm, idx_v, rows_v, sem):
    wid = lax.axis_index("s") * 2 + lax.axis_index("c")
    base = wid * _PER_W

    def body(i, carry):
        off = base + i * _CHUNK
        pltpu.sync_copy(idx_hbm.at[pl.ds(off, _CHUNK)], idx_v)
        pltpu.async_copy(table_hbm.at[idx_v], rows_v, sem).wait()
        pltpu.sync_copy(rows_v, out_hbm.at[pl.ds(off, _CHUNK)])
        return carry

    lax.fori_loop(0, _NCHUNK, body, 0)


def kernel(embedding, indices):
    flat = indices.reshape(_TOTAL)
    out = _gather(embedding, flat)
    return out.reshape(_BATCH, _FIELDS, _EMBED_DIM)


# speedup vs baseline: 1.5754x; 1.0194x over previous
import functools

import jax
import jax.numpy as jnp
from jax import lax
from jax.experimental import pallas as pl
from jax.experimental.pallas import tpu as pltpu
from jax.experimental.pallas import tpu_sc as plsc

_VOCAB = 1000000
_EMBED_DIM = 32
_BATCH = 16384
_FIELDS = 26
_TOTAL = _BATCH * _FIELDS
_NW = 32
_PER_W = _TOTAL // _NW
_CHUNK = 1664
_NCHUNK = _PER_W // _CHUNK
_NBUF = 2

_mesh = plsc.VectorSubcoreMesh(core_axis_name="c", subcore_axis_name="s")


@functools.partial(
    pl.kernel,
    out_type=jax.ShapeDtypeStruct((_TOTAL, _EMBED_DIM), jnp.float32),
    mesh=_mesh,
    scratch_types=[
        pltpu.VMEM((_PER_W,), jnp.int32),
        pltpu.VMEM((_NBUF, _CHUNK, _EMBED_DIM), jnp.float32),
        [pltpu.SemaphoreType.DMA] * _NBUF,
        [pltpu.SemaphoreType.DMA] * _NBUF,
    ],
    compiler_params=pltpu.CompilerParams(use_tc_tiling_on_sc=False),
)
def _gather(table_hbm, idx_hbm, out_hbm, idx_v, rows_v, sems_g, sems_w):
    wid = lax.axis_index("s") * 2 + lax.axis_index("c")
    base = wid * _PER_W

    pltpu.sync_copy(idx_hbm.at[pl.ds(base, _PER_W)], idx_v)

    def gather_start(i, b):
        idx_chunk = idx_v.at[pl.ds(i * _CHUNK, _CHUNK)]
        return pltpu.async_copy(table_hbm.at[idx_chunk], rows_v.at[b], sems_g[b])

    def write_start(i, b):
        dst = out_hbm.at[pl.ds(base + i * _CHUNK, _CHUNK)]
        return pltpu.async_copy(rows_v.at[b], dst, sems_w[b])

    gathers = [gather_start(b, b) for b in range(_NBUF)]
    writes = [None] * _NBUF

    for i in range(_NCHUNK):
        b = i % _NBUF
        gathers[b].wait()
        writes[b] = write_start(i, b)
        nxt = i + _NBUF
        if nxt < _NCHUNK:
            writes[b].wait()
            gathers[b] = gather_start(nxt, b)

    for b in range(_NBUF):
        if writes[b] is not None:
            writes[b].wait()


def kernel(embedding, indices):
    flat = indices.reshape(_TOTAL)
    out = _gather(embedding, flat)
    return out.reshape(_BATCH, _FIELDS, _EMBED_DIM)


# speedup vs baseline: 1.6724x; 1.0615x over previous
import functools

import jax
import jax.numpy as jnp
from jax import lax
from jax.experimental import pallas as pl
from jax.experimental.pallas import tpu as pltpu
from jax.experimental.pallas import tpu_sc as plsc

_VOCAB = 1000000
_EMBED_DIM = 32
_BATCH = 16384
_FIELDS = 26
_TOTAL = _BATCH * _FIELDS
_NW = 32
_PER_W = _TOTAL // _NW
_CHUNK = 1664
_NCHUNK = _PER_W // _CHUNK
_NBUF = 2

_mesh = plsc.VectorSubcoreMesh(core_axis_name="c", subcore_axis_name="s")


@functools.partial(
    pl.kernel,
    out_type=jax.ShapeDtypeStruct((_TOTAL, _EMBED_DIM), jnp.float32),
    mesh=_mesh,
    scratch_types=[
        pltpu.VMEM((_PER_W,), jnp.int32),
        pltpu.VMEM((_NBUF, _CHUNK, _EMBED_DIM), jnp.float32),
        [pltpu.SemaphoreType.DMA] * _NBUF,
        [pltpu.SemaphoreType.DMA] * _NBUF,
    ],
    compiler_params=pltpu.CompilerParams(use_tc_tiling_on_sc=False),
)
def _gather(table_hbm, idx_hbm, out_hbm, idx_v, rows_v, sems_g, sems_w):
    wid = lax.axis_index("s") * 2 + lax.axis_index("c")
    base = wid * _PER_W

    pltpu.sync_copy(idx_hbm.at[pl.ds(base, _PER_W)], idx_v)

    def gather_start(i, b):
        idx_chunk = idx_v.at[pl.ds(i * _CHUNK, _CHUNK)]
        return pltpu.async_copy(table_hbm.at[idx_chunk], rows_v.at[b], sems_g[b])

    def write_start(i, b):
        dst = out_hbm.at[pl.ds(base + i * _CHUNK, _CHUNK)]
        return pltpu.async_copy(rows_v.at[b], dst, sems_w[b])

    gathers = [gather_start(b, b) for b in range(_NBUF)]
    writes = [None] * _NBUF

    for i in range(_NCHUNK):
        b = i % _NBUF
        gathers[b].wait()
        writes[b] = write_start(i, b)
        nxt = i + _NBUF
        if nxt < _NCHUNK:
            writes[b].wait()
            gathers[b] = gather_start(nxt, b)

    for b in range(_NBUF):
        if writes[b] is not None:
            writes[b].wait()


def kernel(embedding, indices):
    flat = indices.T.reshape(_TOTAL)
    out = _gather(embedding, flat)
    return out.reshape(_FIELDS, _BATCH, _EMBED_DIM).transpose(1, 0, 2)


# speedup vs baseline: 1.6746x; 1.0014x over previous
import functools

import jax
import jax.numpy as jnp
from jax import lax
from jax.experimental import pallas as pl
from jax.experimental.pallas import tpu as pltpu
from jax.experimental.pallas import tpu_sc as plsc

_VOCAB = 1000000
_EMBED_DIM = 32
_BATCH = 16384
_FIELDS = 26
_NW = 32
_BSL = _BATCH // _NW
_NBUF = 2

_mesh = plsc.VectorSubcoreMesh(core_axis_name="c", subcore_axis_name="s")


@functools.partial(
    pl.kernel,
    out_type=jax.ShapeDtypeStruct((_FIELDS, _BATCH, _EMBED_DIM), jnp.float32),
    mesh=_mesh,
    scratch_types=[
        pltpu.VMEM((_FIELDS * _BSL,), jnp.int32),
        pltpu.VMEM((_NBUF, _BSL, _EMBED_DIM), jnp.float32),
        pltpu.SemaphoreType.DMA,
        [pltpu.SemaphoreType.DMA] * _NBUF,
        [pltpu.SemaphoreType.DMA] * _NBUF,
    ],
    compiler_params=pltpu.CompilerParams(use_tc_tiling_on_sc=False),
)
def _gather(table_hbm, idx_hbm, out_hbm, idx_fm, rows_v, sem_ix, sems_g, sems_w):
    wid = lax.axis_index("s") * 2 + lax.axis_index("c")
    b0 = wid * _BSL

    for f in range(_FIELDS):
        pltpu.async_copy(
            idx_hbm.at[f, pl.ds(b0, _BSL)], idx_fm.at[pl.ds(f * _BSL, _BSL)], sem_ix
        )
    for f in range(_FIELDS):
        pltpu.make_async_copy(
            idx_hbm.at[0, pl.ds(0, _BSL)], idx_fm.at[pl.ds(0, _BSL)], sem_ix
        ).wait()

    def gather_start(f, b):
        idx_chunk = idx_fm.at[pl.ds(f * _BSL, _BSL)]
        return pltpu.async_copy(table_hbm.at[idx_chunk], rows_v.at[b], sems_g[b])

    def write_start(f, b):
        dst = out_hbm.at[f, pl.ds(b0, _BSL), :]
        return pltpu.async_copy(rows_v.at[b], dst, sems_w[b])

    gathers = [gather_start(b, b) for b in range(_NBUF)]
    writes = [None] * _NBUF

    for f in range(_FIELDS):
        b = f % _NBUF
        gathers[b].wait()
        writes[b] = write_start(f, b)
        nxt = f + _NBUF
        if nxt < _FIELDS:
            writes[b].wait()
            gathers[b] = gather_start(nxt, b)

    for b in range(_NBUF):
        if writes[b] is not None:
            writes[b].wait()


def kernel(embedding, indices):
    out = _gather(embedding, indices.T)
    return out.transpose(1, 0, 2)
